# baseline (speedup 1.0000x reference)
import jax
import jax.numpy as jnp
from jax import lax
from jax.experimental import pallas as pl
from jax.experimental.pallas import tpu as pltpu
from jax.experimental.pallas import tpu_sc as plsc

_N = 50000
_E = 800000
_NC, _NS, _L = 2, 16, 16
_NW = _NC * _NS
_B = 400
_NB = 64
_EPW = _NB * _B
_EPAD = _NW * _EPW
_NPAD = 50176
_STRIPE = _NPAD // _NS
_ZR = 224
_F = 16
_NT = 6
_NBUF = 4
_R = 2000
_G = _N // _R
_EPS = 1e-5


def _sc_agg_body(*refs):
    tabs = refs[0:_NT]
    src_h, dst_h = refs[_NT], refs[_NT + 1]
    outs = refs[_NT + 2:2 * _NT + 2]
    rest = refs[2 * _NT + 2:]
    src_v, dst_v = rest[0], rest[1]
    bufs = rest[2:2 + _NBUF]
    zb, acc = rest[2 + _NBUF], rest[3 + _NBUF]
    sems = rest[4 + _NBUF:]

    c = lax.axis_index("c")
    s = lax.axis_index("s")
    row0 = (c * _NS + s) * _NB
    pltpu.sync_copy(src_h.at[pl.ds(row0, _NB)], src_v)
    pltpu.sync_copy(dst_h.at[pl.ds(row0, _NB)], dst_v)

    z16 = jnp.zeros((_L,), jnp.float32)

    def zrow(r, carry):
        zb[r, :] = z16
        return carry

    lax.fori_loop(0, _ZR, zrow, 0)
    stripe0 = s * _STRIPE

    for t_h, o_h in zip(tabs, outs):
        for k in range(_STRIPE // _ZR):
            pltpu.sync_copy(zb, acc.at[pl.ds(stripe0 + k * _ZR, _ZR)])
        plsc.subcore_barrier()

        for b in range(_NBUF):
            pltpu.async_copy(t_h.at[src_v.at[b]], bufs[b], sems[b])

        def ring(k, carry):
            for b in range(_NBUF):
                j = k * _NBUF + b
                pltpu.make_async_copy(
                    t_h.at[pl.ds(0, _B)], bufs[b], sems[b]).wait()
                pltpu.sync_copy(bufs[b], acc.at[dst_v.at[j]], add=True)

                @pl.when(j + _NBUF < _NB)
                def _():
                    pltpu.async_copy(
                        t_h.at[src_v.at[j + _NBUF]], bufs[b], sems[b])
            return carry

        lax.fori_loop(0, _NB // _NBUF, ring, 0)
        plsc.subcore_barrier()
        pltpu.sync_copy(acc.at[pl.ds(stripe0, _STRIPE)],
                        o_h.at[c, pl.ds(stripe0, _STRIPE)])


def _sc_agg(tabs, src_m, dst_m):
    out = jax.ShapeDtypeStruct((_NC, _NPAD, _F), jnp.float32)
    mesh = plsc.VectorSubcoreMesh(core_axis_name="c", subcore_axis_name="s")
    f = pl.kernel(
        _sc_agg_body,
        out_type=(out,) * _NT,
        mesh=mesh,
        compiler_params=pltpu.CompilerParams(use_tc_tiling_on_sc=False),
        scratch_types=(
            pltpu.VMEM((_NB, _B), jnp.int32),
            pltpu.VMEM((_NB, _B), jnp.int32),
        ) + (pltpu.VMEM((_B, _F), jnp.float32),) * _NBUF + (
            pltpu.VMEM((_ZR, _F), jnp.float32),
            pltpu.VMEM_SHARED((_NPAD, _F), jnp.float32),
        ) + (pltpu.SemaphoreType.DMA,) * _NBUF,
    )
    return f(*tabs, src_m, dst_m)


def _t_mlp_body(t_ref, wt1_ref, bt1_ref, wt2_ref, bt2_ref, o_ref):
    h1 = jax.nn.relu(t_ref[0, 0] * wt1_ref[...] + bt1_ref[...])
    h2 = jax.nn.relu(
        jnp.dot(h1, wt2_ref[...], preferred_element_type=jnp.float32)
        + bt2_ref[...])
    o_ref[...] = jnp.broadcast_to(h2, (8, 32))


def _t_mlp(t_float, wt1, bt1, wt2, bt2):
    return pl.pallas_call(
        _t_mlp_body,
        out_shape=jax.ShapeDtypeStruct((8, 32), jnp.float32),
    )(t_float.reshape(1, 1), wt1, bt1.reshape(1, 32), wt2, bt2.reshape(1, 32))


def _split6(h, refs):
    for i, r in enumerate(refs):
        r[...] = h[:, i * _F:(i + 1) * _F]


def _embed_body(x_ref, y_ref, wx1_ref, bx1_ref, wx2_ref, bx2_ref, emb_ref,
                *out_refs):
    h1 = jax.nn.relu(
        jnp.dot(x_ref[...], wx1_ref[...], preferred_element_type=jnp.float32)
        + bx1_ref[...])
    hx = jax.nn.relu(
        jnp.dot(h1, wx2_ref[...], preferred_element_type=jnp.float32)
        + bx2_ref[...])
    y = y_ref[0, 0, :]
    oh = (y[:, None] == lax.broadcasted_iota(jnp.int32, (_R, 8), 1)
          ).astype(jnp.float32)
    hy = jnp.dot(oh, emb_ref[...], preferred_element_type=jnp.float32)
    _split6(jnp.concatenate([hx, hy], axis=1), out_refs)


def _embed(x, y, wx1, bx1, wx2, bx2, emb):
    o = jax.ShapeDtypeStruct((_N, _F), jnp.float32)
    full = lambda shp: pl.BlockSpec(shp, lambda i: (0,) * len(shp))
    return pl.pallas_call(
        _embed_body,
        grid=(_G,),
        in_specs=[
            pl.BlockSpec((_R, 128), lambda i: (i, 0)),
            pl.BlockSpec((1, 1, _R), lambda i: (i, 0, 0)),
            full((128, 64)), full((1, 64)), full((64, 64)), full((1, 64)),
            full((8, 32)),
        ],
        out_specs=[pl.BlockSpec((_R, _F), lambda i: (i, 0))] * _NT,
        out_shape=(o,) * _NT,
    )(x, y.reshape(_G, 1, _R), wx1, bx1.reshape(1, 64), wx2,
      bx2.reshape(1, 64), emb)


def _ln(h, g, b):
    m = jnp.mean(h, axis=-1, keepdims=True)
    v = jnp.mean((h - m) ** 2, axis=-1, keepdims=True)
    return (h - m) * lax.rsqrt(v + _EPS) * g + b


def _update_body(p0, p1, p2, p3, p4, p5, ht_ref, wx_ref, bx_ref, gx_ref,
                 bex_ref, wy_ref, by_ref, gy_ref, bey_ref, *out_refs):
    a = [p[0] + p[1] for p in (p0, p1, p2, p3, p4, p5)]
    htb = jnp.broadcast_to(ht_ref[0:1, :], (_R, 32))
    hin = jnp.concatenate(a + [htb], axis=1)
    hx = jax.nn.relu(
        jnp.dot(hin, wx_ref[...], preferred_element_type=jnp.float32)
        + bx_ref[...])
    hx = _ln(hx, gx_ref[...], bex_ref[...])
    ay = jnp.concatenate(a[4:6], axis=1)
    hy = jax.nn.relu(
        jnp.dot(ay, wy_ref[...], preferred_element_type=jnp.float32)
        + by_ref[...])
    hy = _ln(hy, gy_ref[...], bey_ref[...])
    _split6(jnp.concatenate([hx, hy], axis=1), out_refs)


def _update(ps, ht, wx, bx, gx, bex, wy, by, gy, bey):
    o = jax.ShapeDtypeStruct((_N, _F), jnp.float32)
    full = lambda shp: pl.BlockSpec(shp, lambda i: (0,) * len(shp))
    pspec = pl.BlockSpec((_NC, _R, _F), lambda i: (0, i, 0))
    return pl.pallas_call(
        _update_body,
        grid=(_G,),
        in_specs=[pspec] * _NT + [
            full((8, 32)),
            full((128, 64)), full((1, 64)), full((1, 64)), full((1, 64)),
            full((32, 32)), full((1, 32)), full((1, 32)), full((1, 32)),
        ],
        out_specs=[pl.BlockSpec((_R, _F), lambda i: (i, 0))] * _NT,
        out_shape=(o,) * _NT,
    )(*ps, ht, wx, bx.reshape(1, 64), gx.reshape(1, 64),
      bex.reshape(1, 64), wy, by.reshape(1, 32), gy.reshape(1, 32),
      bey.reshape(1, 32))


def _final_body(*refs):
    tabs = refs[0:18]
    ht_ref, wo1_ref, bo1_ref, wo2_ref, bo2_ref, o_ref = refs[18:]
    htb = jnp.broadcast_to(ht_ref[0:1, :], (_R, 32))
    hcat = jnp.concatenate([t[...] for t in tabs] + [htb], axis=1)
    z = jax.nn.relu(
        jnp.dot(hcat, wo1_ref[...], preferred_element_type=jnp.float32)
        + bo1_ref[...])
    o_ref[...] = (
        jnp.dot(z, wo2_ref[...], preferred_element_type=jnp.float32)
        + bo2_ref[...])


def _final(tabs, ht, wo1, bo1, wo2, bo2):
    full = lambda shp: pl.BlockSpec(shp, lambda i: (0,) * len(shp))
    blk = pl.BlockSpec((_R, _F), lambda i: (i, 0))
    return pl.pallas_call(
        _final_body,
        grid=(_G,),
        in_specs=[blk] * 18 + [
            full((8, 32)), full((320, 320)), full((1, 320)),
            full((320, 128)), full((1, 128)),
        ],
        out_specs=pl.BlockSpec((_R, 128), lambda i: (i, 0)),
        out_shape=jax.ShapeDtypeStruct((_N, 128), jnp.float32),
    )(*tabs, ht, wo1, bo1.reshape(1, 320), wo2, bo2.reshape(1, 128))


def kernel(t_float, X_t_one_hot, Y_real, edge_index, Wt1, bt1, Wt2, bt2,
           WX1, bX1, WX2, bX2, embY, l0_WX, l0_bX, l0_gX, l0_beX, l0_WY,
           l0_bY, l0_gY, l0_beY, l1_WX, l1_bX, l1_gX, l1_beX, l1_WY, l1_bY,
           l1_gY, l1_beY, Wo1, bo1, Wo2, bo2):
    pad = _EPAD - _E
    src = jnp.concatenate(
        [edge_index[1], jnp.zeros((pad,), jnp.int32)]).reshape(_EPAD // _B, _B)
    dst = jnp.concatenate(
        [edge_index[0], jnp.full((pad,), _N, jnp.int32)]).reshape(
            _EPAD // _B, _B)

    ht = _t_mlp(t_float, Wt1, bt1, Wt2, bt2)
    m0 = _embed(X_t_one_hot, Y_real, WX1, bX1, WX2, bX2, embY)

    p = _sc_agg(m0, src, dst)
    m1 = _update(p, ht, l0_WX, l0_bX, l0_gX, l0_beX,
                 l0_WY, l0_bY, l0_gY, l0_beY)

    p = _sc_agg(m1, src, dst)
    m2 = _update(p, ht, l1_WX, l1_bX, l1_gX, l1_beX,
                 l1_WY, l1_bY, l1_gY, l1_beY)

    tabs = (m0[:4] + m1[:4] + m2[:4] + m0[4:] + m1[4:] + m2[4:])
    logit = _final(tabs, ht, Wo1, bo1, Wo2, bo2)
    return logit.reshape(_N, 16, 8)


# speedup vs baseline: 1.1740x; 1.1740x over previous
import jax
import jax.numpy as jnp
from jax import lax
from jax.experimental import pallas as pl
from jax.experimental.pallas import tpu as pltpu
from jax.experimental.pallas import tpu_sc as plsc

_N = 50000
_E = 800000
_NC, _NS, _L = 2, 16, 16
_NW = _NC * _NS
_B = 128
_NB = 200
_EPW = _NB * _B
_EPAD = _NW * _EPW
_NPAD = 50176
_STRIPE = _NPAD // _NS
_ZR = 112
_F = 32
_NT = 3
_NBUF = 4
_NI = 2 * _NBUF
_R = 2000
_G = _N // _R
_EPS = 1e-5


def _sc_agg_body(*refs):
    tabs = refs[0:_NT]
    src_h, dst_h = refs[_NT], refs[_NT + 1]
    outs = refs[_NT + 2:2 * _NT + 2]
    rest = refs[2 * _NT + 2:]
    bufs = rest[0:_NBUF]
    srcs = rest[_NBUF:_NBUF + _NI]
    dsts = rest[_NBUF + _NI:_NBUF + 2 * _NI]
    zb, acc = rest[_NBUF + 2 * _NI], rest[_NBUF + 2 * _NI + 1]
    gsem = rest[_NBUF + 2 * _NI + 2:2 * _NBUF + 2 * _NI + 2]
    isem = rest[2 * _NBUF + 2 * _NI + 2:]

    c = lax.axis_index("c")
    s = lax.axis_index("s")
    e0 = (c * _NS + s) * _EPW

    z16 = jnp.zeros((_L,), jnp.float32)

    def zrow(r, carry):
        zb[r, 0:16] = z16
        zb[r, 16:32] = z16
        return carry

    lax.fori_loop(0, _ZR, zrow, 0)
    stripe0 = s * _STRIPE

    def load_idx(j, m):
        pltpu.async_copy(src_h.at[pl.ds(e0 + j * _B, _B)], srcs[m], isem[m])
        pltpu.async_copy(dst_h.at[pl.ds(e0 + j * _B, _B)], dsts[m], isem[m])

    def wait_idx(m):
        pltpu.make_async_copy(src_h.at[pl.ds(0, _B)], srcs[m],
                              isem[m]).wait()
        pltpu.make_async_copy(dst_h.at[pl.ds(0, _B)], dsts[m],
                              isem[m]).wait()

    for t_h, o_h in zip(tabs, outs):
        for k in range(_STRIPE // _ZR):
            pltpu.sync_copy(zb, acc.at[pl.ds(stripe0 + k * _ZR, _ZR)])
        plsc.subcore_barrier()

        for m in range(_NI):
            load_idx(m, m)
        for b in range(_NBUF):
            wait_idx(b)
            pltpu.async_copy(t_h.at[srcs[b]], bufs[b], gsem[b])

        def ring(k, carry):
            base = k * _NI
            for p in range(2):
                for b in range(_NBUF):
                    j = base + p * _NBUF + b
                    m = p * _NBUF + b
                    m2 = (m + _NBUF) % _NI
                    pltpu.make_async_copy(
                        t_h.at[pl.ds(0, _B)], bufs[b], gsem[b]).wait()
                    pltpu.sync_copy(bufs[b], acc.at[dsts[m]], add=True)

                    @pl.when(j + _NBUF < _NB)
                    def _():
                        wait_idx(m2)
                        pltpu.async_copy(t_h.at[srcs[m2]], bufs[b], gsem[b])

                    @pl.when(j + _NI < _NB)
                    def _():
                        load_idx(j + _NI, m)
            return carry

        lax.fori_loop(0, _NB // _NI, ring, 0)
        plsc.subcore_barrier()
        pltpu.sync_copy(acc.at[pl.ds(stripe0, _STRIPE)],
                        o_h.at[c, pl.ds(stripe0, _STRIPE)])


def _sc_agg(tabs, src_e, dst_e):
    out = jax.ShapeDtypeStruct((_NC, _NPAD, _F), jnp.float32)
    mesh = plsc.VectorSubcoreMesh(core_axis_name="c", subcore_axis_name="s")
    f = pl.kernel(
        _sc_agg_body,
        out_type=(out,) * _NT,
        mesh=mesh,
        compiler_params=pltpu.CompilerParams(use_tc_tiling_on_sc=False),
        scratch_types=(
            (pltpu.VMEM((_B, _F), jnp.float32),) * _NBUF
            + (pltpu.VMEM((_B,), jnp.int32),) * (2 * _NI)
            + (pltpu.VMEM((_ZR, _F), jnp.float32),
               pltpu.VMEM_SHARED((_NPAD, _F), jnp.float32))
            + (pltpu.SemaphoreType.DMA,) * (_NBUF + _NI)
        ),
    )
    return f(*tabs, src_e, dst_e)


def _t_mlp_body(t_ref, wt1_ref, bt1_ref, wt2_ref, bt2_ref, o_ref):
    h1 = jax.nn.relu(t_ref[0, 0] * wt1_ref[...] + bt1_ref[...])
    h2 = jax.nn.relu(
        jnp.dot(h1, wt2_ref[...], preferred_element_type=jnp.float32)
        + bt2_ref[...])
    o_ref[...] = jnp.broadcast_to(h2, (8, 32))


def _t_mlp(t_float, wt1, bt1, wt2, bt2):
    return pl.pallas_call(
        _t_mlp_body,
        out_shape=jax.ShapeDtypeStruct((8, 32), jnp.float32),
    )(t_float.reshape(1, 1), wt1, bt1.reshape(1, 32), wt2, bt2.reshape(1, 32))


def _embed_body(x_ref, y_ref, wx1_ref, bx1_ref, wx2_ref, bx2_ref, emb_ref,
                t0_ref, t1_ref, t2_ref):
    h1 = jax.nn.relu(
        jnp.dot(x_ref[...], wx1_ref[...], preferred_element_type=jnp.float32)
        + bx1_ref[...])
    hx = jax.nn.relu(
        jnp.dot(h1, wx2_ref[...], preferred_element_type=jnp.float32)
        + bx2_ref[...])
    y = y_ref[0, 0, :]
    oh = (y[:, None] == lax.broadcasted_iota(jnp.int32, (_R, 8), 1)
          ).astype(jnp.float32)
    hy = jnp.dot(oh, emb_ref[...], preferred_element_type=jnp.float32)
    t0_ref[...] = hx[:, :32]
    t1_ref[...] = hx[:, 32:]
    t2_ref[...] = hy


def _embed(x, y, wx1, bx1, wx2, bx2, emb):
    o = jax.ShapeDtypeStruct((_N, _F), jnp.float32)
    full = lambda shp: pl.BlockSpec(shp, lambda i: (0,) * len(shp))
    return pl.pallas_call(
        _embed_body,
        grid=(_G,),
        in_specs=[
            pl.BlockSpec((_R, 128), lambda i: (i, 0)),
            pl.BlockSpec((1, 1, _R), lambda i: (i, 0, 0)),
            full((128, 64)), full((1, 64)), full((64, 64)), full((1, 64)),
            full((8, 32)),
        ],
        out_specs=[pl.BlockSpec((_R, _F), lambda i: (i, 0))] * _NT,
        out_shape=(o,) * _NT,
    )(x, y.reshape(_G, 1, _R), wx1, bx1.reshape(1, 64), wx2,
      bx2.reshape(1, 64), emb)


def _ln(h, g, b):
    m = jnp.mean(h, axis=-1, keepdims=True)
    v = jnp.mean((h - m) ** 2, axis=-1, keepdims=True)
    return (h - m) * lax.rsqrt(v + _EPS) * g + b


def _update_body(p0_ref, p1_ref, p2_ref, ht_ref, wx_ref, bx_ref, gx_ref,
                 bex_ref, wy_ref, by_ref, gy_ref, bey_ref,
                 t0_ref, t1_ref, t2_ref):
    a0 = p0_ref[0] + p0_ref[1]
    a1 = p1_ref[0] + p1_ref[1]
    a2 = p2_ref[0] + p2_ref[1]
    htb = jnp.broadcast_to(ht_ref[0:1, :], (_R, 32))
    hin = jnp.concatenate([a0, a1, a2, htb], axis=1)
    hx = jax.nn.relu(
        jnp.dot(hin, wx_ref[...], preferred_element_type=jnp.float32)
        + bx_ref[...])
    hx = _ln(hx, gx_ref[...], bex_ref[...])
    hy = jax.nn.relu(
        jnp.dot(a2, wy_ref[...], preferred_element_type=jnp.float32)
        + by_ref[...])
    hy = _ln(hy, gy_ref[...], bey_ref[...])
    t0_ref[...] = hx[:, :32]
    t1_ref[...] = hx[:, 32:]
    t2_ref[...] = hy


def _update(ps, ht, wx, bx, gx, bex, wy, by, gy, bey):
    o = jax.ShapeDtypeStruct((_N, _F), jnp.float32)
    full = lambda shp: pl.BlockSpec(shp, lambda i: (0,) * len(shp))
    pspec = pl.BlockSpec((_NC, _R, _F), lambda i: (0, i, 0))
    return pl.pallas_call(
        _update_body,
        grid=(_G,),
        in_specs=[
            pspec, pspec, pspec, full((8, 32)),
            full((128, 64)), full((1, 64)), full((1, 64)), full((1, 64)),
            full((32, 32)), full((1, 32)), full((1, 32)), full((1, 32)),
        ],
        out_specs=[pl.BlockSpec((_R, _F), lambda i: (i, 0))] * _NT,
        out_shape=(o,) * _NT,
    )(*ps, ht, wx, bx.reshape(1, 64), gx.reshape(1, 64),
      bex.reshape(1, 64), wy, by.reshape(1, 32), gy.reshape(1, 32),
      bey.reshape(1, 32))


def _final_body(*refs):
    tabs = refs[0:9]
    ht_ref, wo1_ref, bo1_ref, wo2_ref, bo2_ref, o_ref = refs[9:]
    htb = jnp.broadcast_to(ht_ref[0:1, :], (_R, 32))
    hcat = jnp.concatenate([t[...] for t in tabs] + [htb], axis=1)
    z = jax.nn.relu(
        jnp.dot(hcat, wo1_ref[...], preferred_element_type=jnp.float32)
        + bo1_ref[...])
    o_ref[...] = (
        jnp.dot(z, wo2_ref[...], preferred_element_type=jnp.float32)
        + bo2_ref[...])


def _final(tabs, ht, wo1, bo1, wo2, bo2):
    full = lambda shp: pl.BlockSpec(shp, lambda i: (0,) * len(shp))
    blk = pl.BlockSpec((_R, _F), lambda i: (i, 0))
    return pl.pallas_call(
        _final_body,
        grid=(_G,),
        in_specs=[blk] * 9 + [
            full((8, 32)), full((320, 320)), full((1, 320)),
            full((320, 128)), full((1, 128)),
        ],
        out_specs=pl.BlockSpec((_R, 128), lambda i: (i, 0)),
        out_shape=jax.ShapeDtypeStruct((_N, 128), jnp.float32),
    )(*tabs, ht, wo1, bo1.reshape(1, 320), wo2, bo2.reshape(1, 128))


def kernel(t_float, X_t_one_hot, Y_real, edge_index, Wt1, bt1, Wt2, bt2,
           WX1, bX1, WX2, bX2, embY, l0_WX, l0_bX, l0_gX, l0_beX, l0_WY,
           l0_bY, l0_gY, l0_beY, l1_WX, l1_bX, l1_gX, l1_beX, l1_WY, l1_bY,
           l1_gY, l1_beY, Wo1, bo1, Wo2, bo2):
    pad = _EPAD - _E
    src = jnp.concatenate([edge_index[1], jnp.zeros((pad,), jnp.int32)])
    dst = jnp.concatenate([edge_index[0], jnp.full((pad,), _N, jnp.int32)])

    ht = _t_mlp(t_float, Wt1, bt1, Wt2, bt2)
    m0 = _embed(X_t_one_hot, Y_real, WX1, bX1, WX2, bX2, embY)

    p = _sc_agg(m0, src, dst)
    m1 = _update(p, ht, l0_WX, l0_bX, l0_gX, l0_beX,
                 l0_WY, l0_bY, l0_gY, l0_beY)

    p = _sc_agg(m1, src, dst)
    m2 = _update(p, ht, l1_WX, l1_bX, l1_gX, l1_beX,
                 l1_WY, l1_bY, l1_gY, l1_beY)

    tabs = (m0[0], m0[1], m1[0], m1[1], m2[0], m2[1], m0[2], m1[2], m2[2])
    logit = _final(tabs, ht, Wo1, bo1, Wo2, bo2)
    return logit.reshape(_N, 16, 8)


# speedup vs baseline: 2.6817x; 2.2842x over previous
import jax
import jax.numpy as jnp
from jax import lax
from jax.experimental import pallas as pl
from jax.experimental.pallas import tpu as pltpu
from jax.experimental.pallas import tpu_sc as plsc

_N = 50000
_E = 800000
_NC, _NS, _L = 2, 16, 16
_NW = _NC * _NS
_B = 128
_NB = 200
_EPW = _NB * _B
_EPAD = _NW * _EPW
_NPAD = 50176
_STRIPE = _NPAD // _NS
_ZR = 112
_F = 32
_NT = 3
_NBUF = 4
_NI = 2 * _NBUF
_R = 2000
_G = _N // _R
_EPS = 1e-5


def _sc_agg_body(*refs):
    tabs = refs[0:_NT]
    src_h, dst_h = refs[_NT], refs[_NT + 1]
    outs = refs[_NT + 2:2 * _NT + 2]
    rest = refs[2 * _NT + 2:]
    bufs = rest[0:_NBUF]
    srcs = rest[_NBUF:_NBUF + _NI]
    dsts = rest[_NBUF + _NI:_NBUF + 2 * _NI]
    zb, acc = rest[_NBUF + 2 * _NI], rest[_NBUF + 2 * _NI + 1]
    gsem = rest[_NBUF + 2 * _NI + 2:2 * _NBUF + 2 * _NI + 2]
    isem = rest[2 * _NBUF + 2 * _NI + 2:]

    c = lax.axis_index("c")
    s = lax.axis_index("s")
    e0 = (c * _NS + s) * _EPW

    z16 = jnp.zeros((_L,), jnp.float32)

    def zrow(r, carry):
        zb[r, 0:16] = z16
        zb[r, 16:32] = z16
        return carry

    lax.fori_loop(0, _ZR, zrow, 0)
    stripe0 = s * _STRIPE

    def load_idx(j, m):
        pltpu.async_copy(src_h.at[pl.ds(e0 + j * _B, _B)], srcs[m], isem[m])
        pltpu.async_copy(dst_h.at[pl.ds(e0 + j * _B, _B)], dsts[m], isem[m])

    def wait_idx(m):
        pltpu.make_async_copy(src_h.at[pl.ds(0, _B)], srcs[m],
                              isem[m]).wait()
        pltpu.make_async_copy(dst_h.at[pl.ds(0, _B)], dsts[m],
                              isem[m]).wait()

    for t_h, o_h in zip(tabs, outs):
        for k in range(_STRIPE // _ZR):
            pltpu.sync_copy(zb, acc.at[pl.ds(stripe0 + k * _ZR, _ZR)])
        plsc.subcore_barrier()

        for m in range(_NI):
            load_idx(m, m)
        for b in range(_NBUF):
            wait_idx(b)
            pltpu.async_copy(t_h.at[srcs[b]], bufs[b], gsem[b])

        def ring(k, carry):
            base = k * _NI
            for p in range(2):
                for b in range(_NBUF):
                    j = base + p * _NBUF + b
                    m = p * _NBUF + b
                    m2 = (m + _NBUF) % _NI
                    pltpu.make_async_copy(
                        t_h.at[pl.ds(0, _B)], bufs[b], gsem[b]).wait()
                    pltpu.sync_copy(bufs[b], acc.at[dsts[m]], add=True)

                    @pl.when(j + _NBUF < _NB)
                    def _():
                        wait_idx(m2)
                        pltpu.async_copy(t_h.at[srcs[m2]], bufs[b], gsem[b])

                    @pl.when(j + _NI < _NB)
                    def _():
                        load_idx(j + _NI, m)
            return carry

        lax.fori_loop(0, _NB // _NI, ring, 0)
        plsc.subcore_barrier()
        pltpu.sync_copy(acc.at[pl.ds(stripe0, _STRIPE)],
                        o_h.at[c, pl.ds(stripe0, _STRIPE)])


def _sc_agg(tabs, src_e, dst_e):
    out = jax.ShapeDtypeStruct((_NC, _NPAD, _F), jnp.float32)
    mesh = plsc.VectorSubcoreMesh(core_axis_name="c", subcore_axis_name="s")
    f = pl.kernel(
        _sc_agg_body,
        out_type=(out,) * _NT,
        mesh=mesh,
        compiler_params=pltpu.CompilerParams(use_tc_tiling_on_sc=False),
        scratch_types=(
            (pltpu.VMEM((_B, _F), jnp.float32),) * _NBUF
            + (pltpu.VMEM((_B,), jnp.int32),) * (2 * _NI)
            + (pltpu.VMEM((_ZR, _F), jnp.float32),
               pltpu.VMEM_SHARED((_NPAD, _F), jnp.float32))
            + (pltpu.SemaphoreType.DMA,) * (_NBUF + _NI)
        ),
    )
    return f(*tabs, src_e, dst_e)


def _t_mlp_body(t_ref, wt1_ref, bt1_ref, wt2_ref, bt2_ref, o_ref):
    h1 = jax.nn.relu(t_ref[0, 0] * wt1_ref[...] + bt1_ref[...])
    h2 = jax.nn.relu(
        jnp.dot(h1, wt2_ref[...], preferred_element_type=jnp.float32)
        + bt2_ref[...])
    o_ref[...] = jnp.broadcast_to(h2, (8, 32))


def _t_mlp(t_float, wt1, bt1, wt2, bt2):
    return pl.pallas_call(
        _t_mlp_body,
        out_shape=jax.ShapeDtypeStruct((8, 32), jnp.float32),
    )(t_float.reshape(1, 1), wt1, bt1.reshape(1, 32), wt2, bt2.reshape(1, 32))


def _embed_body(x_ref, y_ref, wx1_ref, bx1_ref, wx2_ref, bx2_ref, emb_ref,
                t0_ref, t1_ref, t2_ref):
    h1 = jax.nn.relu(
        jnp.dot(x_ref[...], wx1_ref[...], preferred_element_type=jnp.float32)
        + bx1_ref[...])
    hx = jax.nn.relu(
        jnp.dot(h1, wx2_ref[...], preferred_element_type=jnp.float32)
        + bx2_ref[...])
    y = y_ref[0, 0, :]
    oh = (y[:, None] == lax.broadcasted_iota(jnp.int32, (_R, 8), 1)
          ).astype(jnp.float32)
    hy = jnp.dot(oh, emb_ref[...], preferred_element_type=jnp.float32)
    t0_ref[...] = hx[:, :32]
    t1_ref[...] = hx[:, 32:]
    t2_ref[...] = hy


def _embed(x, y, wx1, bx1, wx2, bx2, emb):
    o = jax.ShapeDtypeStruct((_N, _F), jnp.float32)
    full = lambda shp: pl.BlockSpec(shp, lambda i: (0,) * len(shp))
    return pl.pallas_call(
        _embed_body,
        grid=(_G,),
        in_specs=[
            pl.BlockSpec((_R, 128), lambda i: (i, 0)),
            pl.BlockSpec((1, 1, _R), lambda i: (i, 0, 0)),
            full((128, 64)), full((1, 64)), full((64, 64)), full((1, 64)),
            full((8, 32)),
        ],
        out_specs=[pl.BlockSpec((_R, _F), lambda i: (i, 0))] * _NT,
        out_shape=(o,) * _NT,
    )(x, y.reshape(_G, 1, _R), wx1, bx1.reshape(1, 64), wx2,
      bx2.reshape(1, 64), emb)


def _ln(h, g, b):
    m = jnp.mean(h, axis=-1, keepdims=True)
    v = jnp.mean((h - m) ** 2, axis=-1, keepdims=True)
    return (h - m) * lax.rsqrt(v + _EPS) * g + b


def _update_body(p0_ref, p1_ref, p2_ref, ht_ref, wx_ref, bx_ref, gx_ref,
                 bex_ref, wy_ref, by_ref, gy_ref, bey_ref,
                 t0_ref, t1_ref, t2_ref):
    a0 = p0_ref[0] + p0_ref[1]
    a1 = p1_ref[0] + p1_ref[1]
    a2 = p2_ref[0] + p2_ref[1]
    htb = jnp.broadcast_to(ht_ref[0:1, :], (_R, 32))
    hin = jnp.concatenate([a0, a1, a2, htb], axis=1)
    hx = jax.nn.relu(
        jnp.dot(hin, wx_ref[...], preferred_element_type=jnp.float32)
        + bx_ref[...])
    hx = _ln(hx, gx_ref[...], bex_ref[...])
    hy = jax.nn.relu(
        jnp.dot(a2, wy_ref[...], preferred_element_type=jnp.float32)
        + by_ref[...])
    hy = _ln(hy, gy_ref[...], bey_ref[...])
    t0_ref[...] = hx[:, :32]
    t1_ref[...] = hx[:, 32:]
    t2_ref[...] = hy


def _update(ps, ht, wx, bx, gx, bex, wy, by, gy, bey):
    o = jax.ShapeDtypeStruct((_N, _F), jnp.float32)
    full = lambda shp: pl.BlockSpec(shp, lambda i: (0,) * len(shp))
    pspec = pl.BlockSpec((_NC, _R, _F), lambda i: (0, i, 0))
    return pl.pallas_call(
        _update_body,
        grid=(_G,),
        in_specs=[
            pspec, pspec, pspec, full((8, 32)),
            full((128, 64)), full((1, 64)), full((1, 64)), full((1, 64)),
            full((32, 32)), full((1, 32)), full((1, 32)), full((1, 32)),
        ],
        out_specs=[pl.BlockSpec((_R, _F), lambda i: (i, 0))] * _NT,
        out_shape=(o,) * _NT,
    )(*ps, ht, wx, bx.reshape(1, 64), gx.reshape(1, 64),
      bex.reshape(1, 64), wy, by.reshape(1, 32), gy.reshape(1, 32),
      bey.reshape(1, 32))


def _final_body(*refs):
    tabs = refs[0:9]
    ht_ref, wo1_ref, bo1_ref, wo2_ref, bo2_ref, o_ref = refs[9:]
    htb = jnp.broadcast_to(ht_ref[0:1, :], (_R, 32))
    hcat = jnp.concatenate([t[...] for t in tabs] + [htb], axis=1)
    z = jax.nn.relu(
        jnp.dot(hcat, wo1_ref[...], preferred_element_type=jnp.float32)
        + bo1_ref[...])
    o_ref[...] = (
        jnp.dot(z, wo2_ref[...], preferred_element_type=jnp.float32)
        + bo2_ref[...])


def _final(tabs, ht, wo1, bo1, wo2, bo2):
    full = lambda shp: pl.BlockSpec(shp, lambda i: (0,) * len(shp))
    blk = pl.BlockSpec((_R, _F), lambda i: (i, 0))
    return pl.pallas_call(
        _final_body,
        grid=(_G,),
        in_specs=[blk] * 9 + [
            full((8, 32)), full((320, 320)), full((1, 320)),
            full((320, 128)), full((1, 128)),
        ],
        out_specs=pl.BlockSpec((_R, 128), lambda i: (i, 0)),
        out_shape=jax.ShapeDtypeStruct((_N, 128), jnp.float32),
    )(*tabs, ht, wo1, bo1.reshape(1, 320), wo2, bo2.reshape(1, 128))


def kernel(t_float, X_t_one_hot, Y_real, edge_index, Wt1, bt1, Wt2, bt2,
           WX1, bX1, WX2, bX2, embY, l0_WX, l0_bX, l0_gX, l0_beX, l0_WY,
           l0_bY, l0_gY, l0_beY, l1_WX, l1_bX, l1_gX, l1_beX, l1_WY, l1_bY,
           l1_gY, l1_beY, Wo1, bo1, Wo2, bo2):
    pad = _EPAD - _E
    pi = jnp.arange(pad, dtype=jnp.int32)
    src = jnp.concatenate([edge_index[1], (pi * 97) % _N])
    dst = jnp.concatenate([edge_index[0], _N + pi % (_NPAD - _N)])

    ht = _t_mlp(t_float, Wt1, bt1, Wt2, bt2)
    m0 = _embed(X_t_one_hot, Y_real, WX1, bX1, WX2, bX2, embY)

    p = _sc_agg(m0, src, dst)
    m1 = _update(p, ht, l0_WX, l0_bX, l0_gX, l0_beX,
                 l0_WY, l0_bY, l0_gY, l0_beY)

    p = _sc_agg(m1, src, dst)
    m2 = _update(p, ht, l1_WX, l1_bX, l1_gX, l1_beX,
                 l1_WY, l1_bY, l1_gY, l1_beY)

    tabs = (m0[0], m0[1], m1[0], m1[1], m2[0], m2[1], m0[2], m1[2], m2[2])
    logit = _final(tabs, ht, Wo1, bo1, Wo2, bo2)
    return logit.reshape(_N, 16, 8)


# speedup vs baseline: 3.1779x; 1.1850x over previous
import jax
import jax.numpy as jnp
from jax import lax
from jax.experimental import pallas as pl
from jax.experimental.pallas import tpu as pltpu
from jax.experimental.pallas import tpu_sc as plsc

_N = 50000
_E = 800000
_NC, _NS, _L = 2, 16, 16
_NW = _NC * _NS
_B = 128
_NB = 200
_EPW = _NB * _B
_EPAD = _NW * _EPW
_NPAD = 50176
_STRIPE = _NPAD // _NS
_ZR = 112
_F = 32
_NT = 3
_NBUF = 4
_NI = 2 * _NBUF
_R = 2000
_G = _N // _R
_EPS = 1e-5


def _sc_agg_body(*refs):
    tabs = refs[0:_NT]
    src_h, dst_h = refs[_NT], refs[_NT + 1]
    o_h = refs[_NT + 2]
    rest = refs[_NT + 3:]
    bufs = rest[0:_NBUF]
    srcs = rest[_NBUF:_NBUF + _NI]
    dsts = rest[_NBUF + _NI:_NBUF + 2 * _NI]
    zb, acc = rest[_NBUF + 2 * _NI], rest[_NBUF + 2 * _NI + 1]
    gsem = rest[_NBUF + 2 * _NI + 2:2 * _NBUF + 2 * _NI + 2]
    isem = rest[2 * _NBUF + 2 * _NI + 2:]

    c = lax.axis_index("c")
    s = lax.axis_index("s")
    e0 = (c * _NS + s) * _EPW

    z16 = jnp.zeros((_L,), jnp.float32)

    def zrow(r, carry):
        zb[r, 0:16] = z16
        zb[r, 16:32] = z16
        return carry

    lax.fori_loop(0, _ZR, zrow, 0)
    stripe0 = s * _STRIPE

    def load_idx(j, m):
        pltpu.async_copy(src_h.at[pl.ds(e0 + j * _B, _B)], srcs[m], isem[m])
        pltpu.async_copy(dst_h.at[pl.ds(e0 + j * _B, _B)], dsts[m], isem[m])

    def wait_idx(m):
        pltpu.make_async_copy(src_h.at[pl.ds(0, _B)], srcs[m],
                              isem[m]).wait()
        pltpu.make_async_copy(dst_h.at[pl.ds(0, _B)], dsts[m],
                              isem[m]).wait()

    for p, t_h in enumerate(tabs):
        for k in range(_STRIPE // _ZR):
            pltpu.sync_copy(zb, acc.at[pl.ds(stripe0 + k * _ZR, _ZR)])
        plsc.subcore_barrier()

        for m in range(_NI):
            load_idx(m, m)
        for b in range(_NBUF):
            wait_idx(b)
            pltpu.async_copy(t_h.at[srcs[b]], bufs[b], gsem[b])

        def ring(k, carry):
            base = k * _NI
            for p in range(2):
                for b in range(_NBUF):
                    j = base + p * _NBUF + b
                    m = p * _NBUF + b
                    m2 = (m + _NBUF) % _NI
                    pltpu.make_async_copy(
                        t_h.at[pl.ds(0, _B)], bufs[b], gsem[b]).wait()
                    pltpu.sync_copy(bufs[b], acc.at[dsts[m]], add=True)

                    @pl.when(j + _NBUF < _NB)
                    def _():
                        wait_idx(m2)
                        pltpu.async_copy(t_h.at[srcs[m2]], bufs[b], gsem[b])

                    @pl.when(j + _NI < _NB)
                    def _():
                        load_idx(j + _NI, m)
            return carry

        lax.fori_loop(0, _NB // _NI, ring, 0)
        plsc.subcore_barrier()
        pltpu.sync_copy(acc.at[pl.ds(stripe0, _STRIPE)],
                        o_h.at[c, pl.ds(stripe0, _STRIPE),
                               pl.ds(p * _F, _F)])


def _sc_agg(tabs, src_e, dst_e):
    out = jax.ShapeDtypeStruct((_NC, _NPAD, 128), jnp.float32)
    mesh = plsc.VectorSubcoreMesh(core_axis_name="c", subcore_axis_name="s")
    f = pl.kernel(
        _sc_agg_body,
        out_type=out,
        mesh=mesh,
        compiler_params=pltpu.CompilerParams(use_tc_tiling_on_sc=False),
        scratch_types=(
            (pltpu.VMEM((_B, _F), jnp.float32),) * _NBUF
            + (pltpu.VMEM((_B,), jnp.int32),) * (2 * _NI)
            + (pltpu.VMEM((_ZR, _F), jnp.float32),
               pltpu.VMEM_SHARED((_NPAD, _F), jnp.float32))
            + (pltpu.SemaphoreType.DMA,) * (_NBUF + _NI)
        ),
    )
    return f(*tabs, src_e, dst_e)


def _t_mlp_body(t_ref, wt1_ref, bt1_ref, wt2_ref, bt2_ref, o_ref):
    h1 = jax.nn.relu(t_ref[0, 0] * wt1_ref[...] + bt1_ref[...])
    h2 = jax.nn.relu(
        jnp.dot(h1, wt2_ref[...], preferred_element_type=jnp.float32)
        + bt2_ref[...])
    o_ref[...] = jnp.broadcast_to(h2, (8, 32))


def _t_mlp(t_float, wt1, bt1, wt2, bt2):
    return pl.pallas_call(
        _t_mlp_body,
        out_shape=jax.ShapeDtypeStruct((8, 32), jnp.float32),
    )(t_float.reshape(1, 1), wt1, bt1.reshape(1, 32), wt2, bt2.reshape(1, 32))


def _embed_body(x_ref, y_ref, wx1_ref, bx1_ref, wx2_ref, bx2_ref, emb_ref,
                t0_ref, t1_ref, t2_ref):
    h1 = jax.nn.relu(
        jnp.dot(x_ref[...], wx1_ref[...], preferred_element_type=jnp.float32)
        + bx1_ref[...])
    hx = jax.nn.relu(
        jnp.dot(h1, wx2_ref[...], preferred_element_type=jnp.float32)
        + bx2_ref[...])
    y = y_ref[0, 0, :]
    oh = (y[:, None] == lax.broadcasted_iota(jnp.int32, (_R, 8), 1)
          ).astype(jnp.float32)
    hy = jnp.dot(oh, emb_ref[...], preferred_element_type=jnp.float32)
    t0_ref[...] = hx[:, :32]
    t1_ref[...] = hx[:, 32:]
    t2_ref[...] = hy


def _embed(x, y, wx1, bx1, wx2, bx2, emb):
    o = jax.ShapeDtypeStruct((_N, _F), jnp.float32)
    full = lambda shp: pl.BlockSpec(shp, lambda i: (0,) * len(shp))
    return pl.pallas_call(
        _embed_body,
        grid=(_G,),
        in_specs=[
            pl.BlockSpec((_R, 128), lambda i: (i, 0)),
            pl.BlockSpec((1, 1, _R), lambda i: (i, 0, 0)),
            full((128, 64)), full((1, 64)), full((64, 64)), full((1, 64)),
            full((8, 32)),
        ],
        out_specs=[pl.BlockSpec((_R, _F), lambda i: (i, 0))] * _NT,
        out_shape=(o,) * _NT,
    )(x, y.reshape(_G, 1, _R), wx1, bx1.reshape(1, 64), wx2,
      bx2.reshape(1, 64), emb)


def _ln(h, g, b):
    m = jnp.mean(h, axis=-1, keepdims=True)
    v = jnp.mean((h - m) ** 2, axis=-1, keepdims=True)
    return (h - m) * lax.rsqrt(v + _EPS) * g + b


def _update_body(p_ref, ht_ref, wx_ref, bx_ref, gx_ref,
                 bex_ref, wy_ref, by_ref, gy_ref, bey_ref,
                 t0_ref, t1_ref, t2_ref):
    a = p_ref[0] + p_ref[1]
    a2 = a[:, 64:96]
    htb = jnp.broadcast_to(ht_ref[0:1, :], (_R, 32))
    hin = jnp.concatenate([a[:, :96], htb], axis=1)
    hx = jax.nn.relu(
        jnp.dot(hin, wx_ref[...], preferred_element_type=jnp.float32)
        + bx_ref[...])
    hx = _ln(hx, gx_ref[...], bex_ref[...])
    hy = jax.nn.relu(
        jnp.dot(a2, wy_ref[...], preferred_element_type=jnp.float32)
        + by_ref[...])
    hy = _ln(hy, gy_ref[...], bey_ref[...])
    t0_ref[...] = hx[:, :32]
    t1_ref[...] = hx[:, 32:]
    t2_ref[...] = hy


def _update(ps, ht, wx, bx, gx, bex, wy, by, gy, bey):
    o = jax.ShapeDtypeStruct((_N, _F), jnp.float32)
    full = lambda shp: pl.BlockSpec(shp, lambda i: (0,) * len(shp))
    pspec = pl.BlockSpec((_NC, _R, 128), lambda i: (0, i, 0))
    return pl.pallas_call(
        _update_body,
        grid=(_G,),
        in_specs=[
            pspec, full((8, 32)),
            full((128, 64)), full((1, 64)), full((1, 64)), full((1, 64)),
            full((32, 32)), full((1, 32)), full((1, 32)), full((1, 32)),
        ],
        out_specs=[pl.BlockSpec((_R, _F), lambda i: (i, 0))] * _NT,
        out_shape=(o,) * _NT,
    )(ps, ht, wx, bx.reshape(1, 64), gx.reshape(1, 64),
      bex.reshape(1, 64), wy, by.reshape(1, 32), gy.reshape(1, 32),
      bey.reshape(1, 32))


def _final_body(*refs):
    tabs = refs[0:9]
    ht_ref, wo1_ref, bo1_ref, wo2_ref, bo2_ref, o_ref = refs[9:]
    htb = jnp.broadcast_to(ht_ref[0:1, :], (_R, 32))
    hcat = jnp.concatenate([t[...] for t in tabs] + [htb], axis=1)
    z = jax.nn.relu(
        jnp.dot(hcat, wo1_ref[...], preferred_element_type=jnp.float32)
        + bo1_ref[...])
    o_ref[...] = (
        jnp.dot(z, wo2_ref[...], preferred_element_type=jnp.float32)
        + bo2_ref[...])


def _final(tabs, ht, wo1, bo1, wo2, bo2):
    full = lambda shp: pl.BlockSpec(shp, lambda i: (0,) * len(shp))
    blk = pl.BlockSpec((_R, _F), lambda i: (i, 0))
    return pl.pallas_call(
        _final_body,
        grid=(_G,),
        in_specs=[blk] * 9 + [
            full((8, 32)), full((320, 320)), full((1, 320)),
            full((320, 128)), full((1, 128)),
        ],
        out_specs=pl.BlockSpec((_R, 128), lambda i: (i, 0)),
        out_shape=jax.ShapeDtypeStruct((_N, 128), jnp.float32),
    )(*tabs, ht, wo1, bo1.reshape(1, 320), wo2, bo2.reshape(1, 128))


def kernel(t_float, X_t_one_hot, Y_real, edge_index, Wt1, bt1, Wt2, bt2,
           WX1, bX1, WX2, bX2, embY, l0_WX, l0_bX, l0_gX, l0_beX, l0_WY,
           l0_bY, l0_gY, l0_beY, l1_WX, l1_bX, l1_gX, l1_beX, l1_WY, l1_bY,
           l1_gY, l1_beY, Wo1, bo1, Wo2, bo2):
    pad = _EPAD - _E
    pi = jnp.arange(pad, dtype=jnp.int32)
    src = jnp.concatenate([edge_index[1], (pi * 97) % _N])
    dst = jnp.concatenate([edge_index[0], _N + pi % (_NPAD - _N)])

    ht = _t_mlp(t_float, Wt1, bt1, Wt2, bt2)
    m0 = _embed(X_t_one_hot, Y_real, WX1, bX1, WX2, bX2, embY)

    p = _sc_agg(m0, src, dst)
    m1 = _update(p, ht, l0_WX, l0_bX, l0_gX, l0_beX,
                 l0_WY, l0_bY, l0_gY, l0_beY)

    p = _sc_agg(m1, src, dst)
    m2 = _update(p, ht, l1_WX, l1_bX, l1_gX, l1_beX,
                 l1_WY, l1_bY, l1_gY, l1_beY)

    tabs = (m0[0], m0[1], m1[0], m1[1], m2[0], m2[1], m0[2], m1[2], m2[2])
    logit = _final(tabs, ht, Wo1, bo1, Wo2, bo2)
    return logit.reshape(_N, 16, 8)
